# Initial kernel scaffold; baseline (speedup 1.0000x reference)
import jax
import jax.numpy as jnp
from jax.experimental import pallas as pl


def kernel(x, edge_index, batch_num_nodes, W1, b1, W2, b2, gamma, beta, fc1_w, fc1_b, fc2_w, fc2_b):
    raise NotImplementedError("write your pallas kernel here")


# speedup vs baseline: 5.4949x; 5.4949x over previous
import functools
import math

import jax
import jax.numpy as jnp
from jax import lax
from jax.experimental import pallas as pl
from jax.experimental.pallas import tpu as pltpu
from jax.experimental.pallas import tpu_sc as plsc

N = 100000
D = 32
E = 1600000
NL = 5
NB = 100
NCLS = 10
HALF = N // 2
NS = 16
LANES = 16
SJ = 4
GRP = SJ * 128
NG = 196
PT = NG * GRP
EPAD = NS * PT
ROWS = EPAD // 128
PTR = PT // 128
FL = 3128
FLL = HALF - (NS - 1) * FL
ACC_ROWS = NS * FL
DW = 16
BN_SCALE = 1.0 / math.sqrt(1.0 + 1e-5)

_sc_mesh = plsc.VectorSubcoreMesh(core_axis_name="c", subcore_axis_name="s")


def _dloc_compute(didx, dlocs, base, j):
    dl = dlocs[j]
    for k2 in range(128 // LANES):
        v = didx[j, pl.ds(k2 * LANES, LANES)]
        loc = v - base
        ok = (loc >= 0) & (loc < HALF)
        dl[pl.ds(k2 * LANES, LANES)] = jnp.where(ok, loc, HALF)


def _agg_body(h_hbm, src_hbm, dst_hbm, zeros_hbm, out_hbm,
              accum, sidx, didx, rows,
              dl0, dl1, dl2, dl3, gsem, ssem):
    dlocs = (dl0, dl1, dl2, dl3)
    cid = lax.axis_index("c")
    sid = lax.axis_index("s")
    base = cid * HALF

    pltpu.sync_copy(zeros_hbm, accum.at[pl.ds(sid * FL, FL)])
    plsc.subcore_barrier()

    def grp(g, carry):
        rb = sid * PTR + g * SJ
        pltpu.sync_copy(src_hbm.at[pl.ds(rb, SJ)], sidx)
        pltpu.sync_copy(dst_hbm.at[pl.ds(rb, SJ)], didx)
        gets = [pltpu.async_copy(h_hbm.at[sidx.at[j]], rows.at[j], gsem)
                for j in range(SJ)]
        for j in range(SJ):
            _dloc_compute(didx, dlocs, base, j)
        for cp in gets:
            cp.wait()
        puts = [pltpu.async_copy(rows.at[j], accum.at[dlocs[j]], ssem,
                                 add=True)
                for j in range(SJ)]
        for cp in puts:
            cp.wait()
        return carry

    lax.fori_loop(0, NG, grp, 0)
    plsc.subcore_barrier()

    @pl.when(sid < NS - 1)
    def _():
        pltpu.sync_copy(accum.at[pl.ds(sid * FL, FL)],
                        out_hbm.at[pl.ds(base + sid * FL, FL)])

    @pl.when(sid == NS - 1)
    def _():
        pltpu.sync_copy(accum.at[pl.ds((NS - 1) * FL, FLL)],
                        out_hbm.at[pl.ds(base + (NS - 1) * FL, FLL)])


_agg_call = functools.partial(
    pl.kernel,
    out_type=jax.ShapeDtypeStruct((N, D), jnp.float32),
    mesh=_sc_mesh,
    compiler_params=pltpu.CompilerParams(use_tc_tiling_on_sc=False),
    scratch_types=[
        pltpu.VMEM_SHARED((ACC_ROWS, D), jnp.float32),
        pltpu.VMEM((SJ, 128), jnp.int32),
        pltpu.VMEM((SJ, 128), jnp.int32),
        pltpu.VMEM((SJ, 128, D), jnp.float32),
    ] + [pltpu.VMEM((128,), jnp.int32) for _ in range(SJ)] + [
        pltpu.SemaphoreType.DMA,
        pltpu.SemaphoreType.DMA,
    ],
)(_agg_body)


def _deg_body(dst_hbm, ones_hbm, zeros_hbm, out_hbm,
              accum, onesv, didx,
              dl0, dl1, dl2, dl3, ssem):
    dlocs = (dl0, dl1, dl2, dl3)
    cid = lax.axis_index("c")
    sid = lax.axis_index("s")
    base = cid * HALF

    pltpu.sync_copy(zeros_hbm, accum.at[pl.ds(sid * FL, FL)])
    pltpu.sync_copy(ones_hbm, onesv)
    plsc.subcore_barrier()

    def grp(g, carry):
        rb = sid * PTR + g * SJ
        pltpu.sync_copy(dst_hbm.at[pl.ds(rb, SJ)], didx)
        for j in range(SJ):
            _dloc_compute(didx, dlocs, base, j)
        puts = [pltpu.async_copy(onesv, accum.at[dlocs[j]], ssem, add=True)
                for j in range(SJ)]
        for cp in puts:
            cp.wait()
        return carry

    lax.fori_loop(0, NG, grp, 0)
    plsc.subcore_barrier()

    @pl.when(sid < NS - 1)
    def _():
        pltpu.sync_copy(accum.at[pl.ds(sid * FL, FL)],
                        out_hbm.at[pl.ds(base + sid * FL, FL)])

    @pl.when(sid == NS - 1)
    def _():
        pltpu.sync_copy(accum.at[pl.ds((NS - 1) * FL, FLL)],
                        out_hbm.at[pl.ds(base + (NS - 1) * FL, FLL)])


_deg_call = functools.partial(
    pl.kernel,
    out_type=jax.ShapeDtypeStruct((N, DW), jnp.float32),
    mesh=_sc_mesh,
    compiler_params=pltpu.CompilerParams(use_tc_tiling_on_sc=False),
    scratch_types=[
        pltpu.VMEM_SHARED((ACC_ROWS, DW), jnp.float32),
        pltpu.VMEM((128, DW), jnp.float32),
        pltpu.VMEM((SJ, 128), jnp.int32),
    ] + [pltpu.VMEM((128,), jnp.int32) for _ in range(SJ)] + [
        pltpu.SemaphoreType.DMA,
    ],
)(_deg_body)


BLK = 2000


def _dense_body(h_ref, agg_ref, deg_ref, w1_ref, b1_ref, w2_ref, b2_ref,
                gm_ref, bt_ref, o_ref):
    deg = jnp.maximum(deg_ref[:, 0:1], 1.0)
    rst = h_ref[...] + agg_ref[...] / deg
    u = jnp.maximum(
        jnp.dot(rst, w1_ref[...], preferred_element_type=jnp.float32)
        + b1_ref[...], 0.0)
    y = jnp.dot(u, w2_ref[...], preferred_element_type=jnp.float32) \
        + b2_ref[...]
    o_ref[...] = gm_ref[...] * (jnp.maximum(y, 0.0) * BN_SCALE) + bt_ref[...]


_dense_call = pl.pallas_call(
    _dense_body,
    grid=(N // BLK,),
    in_specs=[
        pl.BlockSpec((BLK, D), lambda i: (i, 0)),
        pl.BlockSpec((BLK, D), lambda i: (i, 0)),
        pl.BlockSpec((BLK, DW), lambda i: (i, 0)),
        pl.BlockSpec((D, D), lambda i: (0, 0)),
        pl.BlockSpec((1, D), lambda i: (0, 0)),
        pl.BlockSpec((D, D), lambda i: (0, 0)),
        pl.BlockSpec((1, D), lambda i: (0, 0)),
        pl.BlockSpec((1, D), lambda i: (0, 0)),
        pl.BlockSpec((1, D), lambda i: (0, 0)),
    ],
    out_specs=pl.BlockSpec((BLK, D), lambda i: (i, 0)),
    out_shape=jax.ShapeDtypeStruct((N, D), jnp.float32),
)


def _gather_body(idx_ref, h_ref, o_ref):
    o_ref[...] = h_ref[...]


_gather_call = pl.pallas_call(
    _gather_body,
    grid_spec=pltpu.PrefetchScalarGridSpec(
        num_scalar_prefetch=1,
        grid=(NB,),
        in_specs=[pl.BlockSpec((1, 1, D), lambda i, idx: (idx[i], 0, 0))],
        out_specs=pl.BlockSpec((1, 1, D), lambda i, idx: (i, 0, 0)),
    ),
    out_shape=jax.ShapeDtypeStruct((NB, 1, D), jnp.float32),
)


def _head_body(g_ref, w1_ref, b1_ref, w2_ref, b2_ref, o_ref):
    g1 = jnp.maximum(
        jnp.dot(g_ref[...], w1_ref[...], preferred_element_type=jnp.float32)
        + b1_ref[...], 0.0)
    logits = jnp.dot(g1, w2_ref[...], preferred_element_type=jnp.float32) \
        + b2_ref[...]
    m = jnp.max(logits, axis=-1, keepdims=True)
    lse = jnp.log(jnp.sum(jnp.exp(logits - m), axis=-1, keepdims=True)) + m
    o_ref[...] = logits - lse


_head_call = pl.pallas_call(
    _head_body,
    out_shape=jax.ShapeDtypeStruct((NB, NCLS), jnp.float32),
)


def kernel(x, edge_index, batch_num_nodes, W1, b1, W2, b2, gamma, beta,
           fc1_w, fc1_b, fc2_w, fc2_b):
    src = edge_index[0]
    dst = edge_index[1]
    pad = EPAD - E
    src2 = jnp.concatenate(
        [src, jnp.zeros((pad,), jnp.int32)]).reshape(ROWS, 128)
    dst2 = jnp.concatenate(
        [dst, jnp.full((pad,), -1, jnp.int32)]).reshape(ROWS, 128)
    zeros32 = jnp.zeros((FL, D), jnp.float32)
    zeros16 = jnp.zeros((FL, DW), jnp.float32)
    ones16 = jnp.ones((128, DW), jnp.float32)

    degf = _deg_call(dst2, ones16, zeros16)

    h = x
    for i in range(NL):
        agg = _agg_call(h, src2, dst2, zeros32)
        h = _dense_call(h, agg, degf, W1[i], b1[i].reshape(1, D), W2[i],
                        b2[i].reshape(1, D), gamma[i].reshape(1, D),
                        beta[i].reshape(1, D))

    idx = (jnp.cumsum(batch_num_nodes) - 1).astype(jnp.int32)
    g = _gather_call(idx, h.reshape(N, 1, D)).reshape(NB, D)
    return _head_call(g, fc1_w, fc1_b.reshape(1, D), fc2_w,
                      fc2_b.reshape(1, NCLS))
